# baseline (speedup 1.0000x reference)
import math

import jax
import jax.numpy as jnp
import numpy as np
from jax.experimental import pallas as pl

T = 2048
IN_DIM = 1024
KV_LORA = 512
Q_LORA = 768
QK_ROPE = 64
QK_NOPE = 128
V_HEAD = 128
NH = 16
IH = 4
IHD = 32
IRD = 16
IK = 512
HD = QK_NOPE + QK_ROPE
KV_B = NH * (QK_NOPE + V_HEAD)

BT = 128
BQ = 256
BA = 256
NEG = -1e9
SCALE = 1.0 / math.sqrt(HD)


def _rope_tables(dim):
    freqs = 1.0 / (10000.0 ** (np.arange(0, dim, 2, dtype=np.float64) / dim))
    ang = np.outer(np.arange(T, dtype=np.float64), freqs)
    cos = np.cos(ang)
    sin = np.sin(ang)
    C = np.empty((T, dim), np.float32)
    S = np.empty((T, dim), np.float32)
    C[:, 0::2] = cos
    C[:, 1::2] = cos
    S[:, 0::2] = -sin
    S[:, 1::2] = sin
    return C, S


_QC_NP, _QS_NP = _rope_tables(QK_ROPE)
_IC_NP, _IS_NP = _rope_tables(IRD)

_SEG_NP = np.zeros((KV_B, 2 * NH), np.float32)
for _h in range(2 * NH):
    _SEG_NP[_h * QK_NOPE:(_h + 1) * QK_NOPE, _h] = 1.0


def _rot(x, c, s):
    lane = jax.lax.broadcasted_iota(jnp.int32, x.shape, 1)
    even = (lane % 2) == 0
    swap = jnp.where(even, jnp.roll(x, -1, axis=1), jnp.roll(x, 1, axis=1))
    return x * c + swap * s


def _ln_full(z, g, b, eps=1e-5):
    m = jnp.mean(z, axis=1, keepdims=True)
    v = jnp.mean((z - m) ** 2, axis=1, keepdims=True)
    return (z - m) / jnp.sqrt(v + eps) * g + b


def _dot(a, b):
    return jnp.dot(a, b, preferred_element_type=jnp.float32)


def _dot_t(a, b):
    return jax.lax.dot_general(a, b, (((1,), (1,)), ((), ())),
                               preferred_element_type=jnp.float32)


def _proj_body(x_ref, x16_ref, qc_ref, qs_ref, ic_ref, is_ref, seg_ref,
               wqa_ref, bqa_ref, wqb_ref, bqb_ref, qg_ref, qb_ref,
               wkva_ref, bkva_ref, kvg_ref, kvb_ref,
               wkvb_ref, bkvb_ref, gknv_ref, bknv_ref,
               wiq_ref, biq_ref, iqg_ref, iqb_ref,
               wik_ref, bik_ref, ikg_ref, ikb_ref,
               wiw_ref, biw_ref, iwg_ref, iwb_ref,
               qn_out, qr_out, knv_out, kr_out, iq_out, ik_out, iw_out):
    x = x_ref[...]
    x16 = x16_ref[...]
    qc = qc_ref[...]
    qs = qs_ref[...]
    ic = ic_ref[...]
    isn = is_ref[...]

    qa = _dot_t(x16, wqa_ref[...]) + bqa_ref[...]
    q = _ln_full(_dot_t(qa.astype(jnp.bfloat16), wqb_ref[...]) + bqb_ref[...],
                 qg_ref[...], qb_ref[...])
    for h in range(NH):
        qn_out[:, h * QK_NOPE:(h + 1) * QK_NOPE] = (
            (q[:, h * HD:h * HD + QK_NOPE] * SCALE).astype(jnp.bfloat16))
        rope = q[:, h * HD + QK_NOPE:(h + 1) * HD]
        qr_out[h, :, :] = (_rot(rope, qc, qs) * SCALE).astype(jnp.bfloat16)

    kvf = _ln_full(_dot_t(x, wkva_ref[...]) + bkva_ref[...],
                   kvg_ref[...], kvb_ref[...])
    kv_lat = kvf[:, :KV_LORA]
    kr_out[...] = _rot(kvf[:, KV_LORA:], qc, qs).astype(jnp.bfloat16)

    seg = seg_ref[...]
    z = _dot_t(kv_lat.astype(jnp.bfloat16), wkvb_ref[...]) + bkvb_ref[...]
    m = _dot(z, seg) * (1.0 / QK_NOPE)
    mb = _dot_t(m, seg)
    q2 = _dot(z * z, seg) * (1.0 / QK_NOPE)
    q2b = _dot_t(q2, seg)
    var = q2b - mb * mb
    knv_out[...] = ((z - mb) / jnp.sqrt(var + 1e-5) * gknv_ref[...]
                    + bknv_ref[...]).astype(jnp.bfloat16)

    qf = _ln_full(_dot_t(x, wiq_ref[...]) + biq_ref[...],
                  iqg_ref[...], iqb_ref[...])
    parts = []
    for h in range(IH):
        nope = qf[:, h * (IHD - IRD):(h + 1) * (IHD - IRD)]
        rope = qf[:, IH * (IHD - IRD) + h * IRD:IH * (IHD - IRD) + (h + 1) * IRD]
        parts.append(nope)
        parts.append(_rot(rope, ic, isn))
    iq_out[...] = jnp.concatenate(parts, axis=1)

    kf = _ln_full(_dot_t(kv_lat, wik_ref[...]) + bik_ref[...],
                  ikg_ref[...], ikb_ref[...])
    ik_out[...] = jnp.concatenate(
        [kf[:, :IHD - IRD], _rot(kf[:, IHD - IRD:], ic, isn)], axis=1)

    iw_out[...] = _ln_full(_dot_t(x, wiw_ref[...]) + biw_ref[...],
                           iwg_ref[...], iwb_ref[...])


def _select_body(iq_ref, ik_ref, iw_ref, mask_ref, sel_out):
    iq = iq_ref[...]
    ik = ik_ref[...]
    iw = iw_ref[...]
    msk = mask_ref[...]

    acc = jnp.zeros((BQ, T), jnp.float32)
    for h in range(IH):
        qh = iq[:, h * IHD:(h + 1) * IHD]
        sh = _dot_t(qh, ik)
        acc = acc + iw[:, h:h + 1] * jnp.maximum(sh, 0.0)

    sc = jnp.where(msk == 0, NEG, acc)
    bits = jax.lax.bitcast_convert_type(sc, jnp.int32)
    imin = jnp.int32(-2147483648)
    keys = jnp.where(bits >= 0, bits, ~bits ^ imin)

    lo = jnp.full((BQ, 1), imin, jnp.int32)
    hi = jnp.full((BQ, 1), jnp.int32(2147483647), jnp.int32)

    def body(_, carry):
        lo, hi = carry
        x = lo ^ hi
        mid = (lo & hi) + (x >> 1) + (x & 1)
        cnt = jnp.sum((keys >= mid).astype(jnp.int32), axis=1, keepdims=True)
        ge = cnt >= IK
        return jnp.where(ge, mid, lo), jnp.where(ge, hi, mid - 1)

    lo, hi = jax.lax.fori_loop(0, 32, body, (lo, hi))
    thr = lo

    gt = keys > thr
    eq = keys == thr
    need = IK - jnp.sum(gt.astype(jnp.int32), axis=1, keepdims=True)
    c = eq.astype(jnp.int32)
    lane = jax.lax.broadcasted_iota(jnp.int32, (BQ, T), 1)
    for shf in (1, 2, 4, 8, 16, 32, 64, 128, 256, 512, 1024):
        c = c + jnp.where(lane >= shf, jnp.roll(c, shf, axis=1), 0)
    selected = gt | (eq & (c <= need))
    keep = selected & (msk != 0)
    sel_out[...] = keep.astype(jnp.int8)


def _attn_body(qn_ref, qr_ref, kn_ref, kr_ref, v_ref, sel_ref, o_out):
    qn = qn_ref[...]
    qr = qr_ref[0, :, :]
    kn = kn_ref[...]
    kr = kr_ref[...]
    v = v_ref[...]
    sel = sel_ref[...]

    s = _dot_t(qn, kn) + _dot_t(qr, kr)
    s = jnp.where(sel != 0, s, NEG)
    m = jnp.max(s, axis=1, keepdims=True)
    p = jnp.exp(s - m)
    l = jnp.sum(p, axis=1, keepdims=True)
    o_out[...] = (_dot(p.astype(jnp.bfloat16), v) / l).astype(jnp.bfloat16)


def _out_body(o_ref, wo_ref, bo_ref, y_out):
    y_out[...] = _dot_t(o_ref[...], wo_ref[...]) + bo_ref[...]


def _row(w):
    return w.reshape(1, -1)


@jax.jit
def kernel(x, mask, params):
    p = params
    x2 = x.reshape(T, IN_DIM)
    x16 = x2.astype(jnp.bfloat16)
    mask2 = mask.reshape(T, T)

    qc = jnp.asarray(_QC_NP)
    qs = jnp.asarray(_QS_NP)
    ic = jnp.asarray(_IC_NP)
    isn = jnp.asarray(_IS_NP)
    seg = jnp.asarray(_SEG_NP)

    wqa16 = p["Wq_a_w"].astype(jnp.bfloat16)
    wqb16 = p["Wq_b_w"].astype(jnp.bfloat16)
    wkvb16 = p["Wkv_b_w"].astype(jnp.bfloat16)
    wo16 = p["Wo_w"].astype(jnp.bfloat16)
    gknv = _row(jnp.tile(jnp.concatenate([p["k_ln_g"], p["v_ln_g"]]), NH))
    bknv = _row(jnp.tile(jnp.concatenate([p["k_ln_b"], p["v_ln_b"]]), NH))

    nbt = T // BT
    full = lambda shape: pl.BlockSpec(shape, lambda i: (0,) * len(shape))
    rowblk = lambda w: pl.BlockSpec((BT, w), lambda i: (i, 0))

    qn, qr, knv, kr, iqf, ikf, iwf = pl.pallas_call(
        _proj_body,
        grid=(nbt,),
        in_specs=[
            rowblk(IN_DIM), rowblk(IN_DIM),
            rowblk(QK_ROPE), rowblk(QK_ROPE),
            rowblk(IRD), rowblk(IRD), full((KV_B, 2 * NH)),
            full((Q_LORA, IN_DIM)), full((1, Q_LORA)),
            full((NH * HD, Q_LORA)), full((1, NH * HD)),
            full((1, NH * HD)), full((1, NH * HD)),
            full((KV_LORA + QK_ROPE, IN_DIM)), full((1, KV_LORA + QK_ROPE)),
            full((1, KV_LORA + QK_ROPE)), full((1, KV_LORA + QK_ROPE)),
            full((KV_B, KV_LORA)), full((1, KV_B)),
            full((1, KV_B)), full((1, KV_B)),
            full((IH * IHD, IN_DIM)), full((1, IH * IHD)),
            full((1, IH * IHD)), full((1, IH * IHD)),
            full((IHD, KV_LORA)), full((1, IHD)),
            full((1, IHD)), full((1, IHD)),
            full((IH, IN_DIM)), full((1, IH)),
            full((1, IH)), full((1, IH)),
        ],
        out_specs=[
            rowblk(NH * QK_NOPE),
            pl.BlockSpec((NH, BT, QK_ROPE), lambda i: (0, i, 0)),
            rowblk(KV_B), rowblk(QK_ROPE),
            rowblk(IH * IHD), rowblk(IHD), rowblk(IH),
        ],
        out_shape=[
            jax.ShapeDtypeStruct((T, NH * QK_NOPE), jnp.bfloat16),
            jax.ShapeDtypeStruct((NH, T, QK_ROPE), jnp.bfloat16),
            jax.ShapeDtypeStruct((T, KV_B), jnp.bfloat16),
            jax.ShapeDtypeStruct((T, QK_ROPE), jnp.bfloat16),
            jax.ShapeDtypeStruct((T, IH * IHD), jnp.float32),
            jax.ShapeDtypeStruct((T, IHD), jnp.float32),
            jax.ShapeDtypeStruct((T, IH), jnp.float32),
        ],
    )(x2, x16, qc, qs, ic, isn, seg,
      wqa16, _row(p["Wq_a_b"]), wqb16, _row(p["Wq_b_b"]),
      _row(p["q_ln_g"]), _row(p["q_ln_b"]),
      p["Wkv_a_w"], _row(p["Wkv_a_b"]), _row(p["kv_ln_g"]), _row(p["kv_ln_b"]),
      wkvb16, _row(p["Wkv_b_b"]), gknv, bknv,
      p["iWq_w"], _row(p["iWq_b"]), _row(p["iq_ln_g"]), _row(p["iq_ln_b"]),
      p["iWk_w"], _row(p["iWk_b"]), _row(p["ik_ln_g"]), _row(p["ik_ln_b"]),
      p["iWw_w"], _row(p["iWw_b"]), _row(p["iw_ln_g"]), _row(p["iw_ln_b"]))

    nbq = T // BQ
    sel = pl.pallas_call(
        _select_body,
        grid=(nbq,),
        in_specs=[
            pl.BlockSpec((BQ, IH * IHD), lambda i: (i, 0)),
            pl.BlockSpec((T, IHD), lambda i: (0, 0)),
            pl.BlockSpec((BQ, IH), lambda i: (i, 0)),
            pl.BlockSpec((BQ, T), lambda i: (i, 0)),
        ],
        out_specs=pl.BlockSpec((BQ, T), lambda i: (i, 0)),
        out_shape=jax.ShapeDtypeStruct((T, T), jnp.int8),
    )(iqf, ikf, iwf, mask2)

    nba = T // BA
    o = pl.pallas_call(
        _attn_body,
        grid=(NH, nba),
        in_specs=[
            pl.BlockSpec((BA, QK_NOPE), lambda h, i: (i, h)),
            pl.BlockSpec((1, BA, QK_ROPE), lambda h, i: (h, i, 0)),
            pl.BlockSpec((T, QK_NOPE), lambda h, i: (0, 2 * h)),
            pl.BlockSpec((T, QK_ROPE), lambda h, i: (0, 0)),
            pl.BlockSpec((T, V_HEAD), lambda h, i: (0, 2 * h + 1)),
            pl.BlockSpec((BA, T), lambda h, i: (i, 0)),
        ],
        out_specs=pl.BlockSpec((BA, V_HEAD), lambda h, i: (i, h)),
        out_shape=jax.ShapeDtypeStruct((T, NH * V_HEAD), jnp.bfloat16),
    )(qn, qr, knv, kr, knv, sel)

    y = pl.pallas_call(
        _out_body,
        grid=(nba,),
        in_specs=[
            pl.BlockSpec((BA, NH * V_HEAD), lambda i: (i, 0)),
            pl.BlockSpec((IN_DIM, NH * V_HEAD), lambda i: (0, 0)),
            pl.BlockSpec((1, IN_DIM), lambda i: (0, 0)),
        ],
        out_specs=pl.BlockSpec((BA, IN_DIM), lambda i: (i, 0)),
        out_shape=jax.ShapeDtypeStruct((T, IN_DIM), jnp.float32),
    )(o, wo16, _row(p["Wo_b"]))

    return y.reshape(1, T, IN_DIM)


# speedup vs baseline: 1.0741x; 1.0741x over previous
import math

import jax
import jax.numpy as jnp
import numpy as np
from jax.experimental import pallas as pl

T = 2048
IN_DIM = 1024
KV_LORA = 512
Q_LORA = 768
QK_ROPE = 64
QK_NOPE = 128
V_HEAD = 128
NH = 16
IH = 4
IHD = 32
IRD = 16
IK = 512
HD = QK_NOPE + QK_ROPE
KV_B = NH * (QK_NOPE + V_HEAD)

BT = 256
BQ = 256
BA = 256
NEG = -1e9
SCALE = 1.0 / math.sqrt(HD)


def _rope_tables(dim):
    freqs = 1.0 / (10000.0 ** (np.arange(0, dim, 2, dtype=np.float64) / dim))
    ang = np.outer(np.arange(T, dtype=np.float64), freqs)
    cos = np.cos(ang)
    sin = np.sin(ang)
    C = np.empty((T, dim), np.float32)
    S = np.empty((T, dim), np.float32)
    C[:, 0::2] = cos
    C[:, 1::2] = cos
    S[:, 0::2] = -sin
    S[:, 1::2] = sin
    return C, S


_QC_NP, _QS_NP = _rope_tables(QK_ROPE)
_IC_NP, _IS_NP = _rope_tables(IRD)

_SEG_NP = np.zeros((KV_B, 2 * NH), np.float32)
for _h in range(2 * NH):
    _SEG_NP[_h * QK_NOPE:(_h + 1) * QK_NOPE, _h] = 1.0


def _rot(x, c, s):
    lane = jax.lax.broadcasted_iota(jnp.int32, x.shape, 1)
    even = (lane % 2) == 0
    swap = jnp.where(even, jnp.roll(x, -1, axis=1), jnp.roll(x, 1, axis=1))
    return x * c + swap * s


def _ln_full(z, g, b, eps=1e-5):
    m = jnp.mean(z, axis=1, keepdims=True)
    v = jnp.mean((z - m) ** 2, axis=1, keepdims=True)
    return (z - m) / jnp.sqrt(v + eps) * g + b


def _dot(a, b):
    return jnp.dot(a, b, preferred_element_type=jnp.float32)


def _dot_t(a, b):
    return jax.lax.dot_general(a, b, (((1,), (1,)), ((), ())),
                               preferred_element_type=jnp.float32)


def _proj_body(x_ref, x16_ref, qc_ref, qs_ref, ic_ref, is_ref, seg_ref,
               wqa_ref, bqa_ref, wqb_ref, bqb_ref, qg_ref, qb_ref,
               wkva_ref, bkva_ref, kvg_ref, kvb_ref,
               wkvb_ref, bkvb_ref, gknv_ref, bknv_ref,
               wiq_ref, biq_ref, iqg_ref, iqb_ref,
               wik_ref, bik_ref, ikg_ref, ikb_ref,
               wiw_ref, biw_ref, iwg_ref, iwb_ref,
               qn_out, qr_out, knv_out, kr_out, iq_out, ik_out, iw_out):
    x = x_ref[...]
    x16 = x16_ref[...]
    qc = qc_ref[...]
    qs = qs_ref[...]
    ic = ic_ref[...]
    isn = is_ref[...]

    qa = _dot_t(x16, wqa_ref[...]) + bqa_ref[...]
    q = _ln_full(_dot_t(qa.astype(jnp.bfloat16), wqb_ref[...]) + bqb_ref[...],
                 qg_ref[...], qb_ref[...])
    for h in range(NH):
        qn_out[:, h * QK_NOPE:(h + 1) * QK_NOPE] = (
            (q[:, h * HD:h * HD + QK_NOPE] * SCALE).astype(jnp.bfloat16))
        rope = q[:, h * HD + QK_NOPE:(h + 1) * HD]
        qr_out[h, :, :] = (_rot(rope, qc, qs) * SCALE).astype(jnp.bfloat16)

    kvf = _ln_full(_dot_t(x, wkva_ref[...]) + bkva_ref[...],
                   kvg_ref[...], kvb_ref[...])
    kv_lat = kvf[:, :KV_LORA]
    kr_out[...] = _rot(kvf[:, KV_LORA:], qc, qs).astype(jnp.bfloat16)

    seg = seg_ref[...]
    z = _dot_t(kv_lat.astype(jnp.bfloat16), wkvb_ref[...]) + bkvb_ref[...]
    m = _dot(z, seg) * (1.0 / QK_NOPE)
    mb = _dot_t(m, seg)
    q2 = _dot(z * z, seg) * (1.0 / QK_NOPE)
    q2b = _dot_t(q2, seg)
    var = q2b - mb * mb
    knv_out[...] = ((z - mb) / jnp.sqrt(var + 1e-5) * gknv_ref[...]
                    + bknv_ref[...]).astype(jnp.bfloat16)

    qf = _ln_full(_dot_t(x, wiq_ref[...]) + biq_ref[...],
                  iqg_ref[...], iqb_ref[...])
    parts = []
    for h in range(IH):
        nope = qf[:, h * (IHD - IRD):(h + 1) * (IHD - IRD)]
        rope = qf[:, IH * (IHD - IRD) + h * IRD:IH * (IHD - IRD) + (h + 1) * IRD]
        parts.append(nope)
        parts.append(_rot(rope, ic, isn))
    iq_out[...] = jnp.concatenate(parts, axis=1)

    kf = _ln_full(_dot_t(kv_lat, wik_ref[...]) + bik_ref[...],
                  ikg_ref[...], ikb_ref[...])
    ik_out[...] = jnp.concatenate(
        [kf[:, :IHD - IRD], _rot(kf[:, IHD - IRD:], ic, isn)], axis=1)

    iw_out[...] = _ln_full(_dot_t(x, wiw_ref[...]) + biw_ref[...],
                           iwg_ref[...], iwb_ref[...])


def _select_body(iq_ref, ik_ref, iw_ref, mask_ref, sel_out):
    iq = iq_ref[...]
    ik = ik_ref[...]
    iw = iw_ref[...]
    msk = mask_ref[...]

    acc = jnp.zeros((BQ, T), jnp.float32)
    for h in range(IH):
        qh = iq[:, h * IHD:(h + 1) * IHD]
        sh = _dot_t(qh, ik)
        acc = acc + iw[:, h:h + 1] * jnp.maximum(sh, 0.0)

    sc = jnp.where(msk == 0, NEG, acc)
    bits = jax.lax.bitcast_convert_type(sc, jnp.int32)
    imin = jnp.int32(-2147483648)
    keys = jnp.where(bits >= 0, bits, ~bits ^ imin)

    lo = jnp.full((BQ, 1), imin, jnp.int32)
    hi = jnp.full((BQ, 1), jnp.int32(2147483647), jnp.int32)

    def body(_, carry):
        lo, hi = carry
        x = lo ^ hi
        mid = (lo & hi) + (x >> 1) + (x & 1)
        cnt = jnp.sum((keys >= mid).astype(jnp.int32), axis=1, keepdims=True)
        ge = cnt >= IK
        return jnp.where(ge, mid, lo), jnp.where(ge, hi, mid - 1)

    lo, hi = jax.lax.fori_loop(0, 32, body, (lo, hi))
    thr = lo

    gt = keys > thr
    eq = keys == thr
    need = IK - jnp.sum(gt.astype(jnp.int32), axis=1, keepdims=True)
    c = eq.astype(jnp.int32)
    lane = jax.lax.broadcasted_iota(jnp.int32, (BQ, T), 1)
    for shf in (1, 2, 4, 8, 16, 32, 64, 128, 256, 512, 1024):
        c = c + jnp.where(lane >= shf, jnp.roll(c, shf, axis=1), 0)
    selected = gt | (eq & (c <= need))
    keep = selected & (msk != 0)
    sel_out[...] = keep.astype(jnp.int8)


def _attn_body(qn_ref, qr_ref, kn_ref, kr_ref, v_ref, sel_ref, o_out):
    qn = qn_ref[...]
    qr = qr_ref[0, :, :]
    kn = kn_ref[...]
    kr = kr_ref[...]
    v = v_ref[...]
    sel = sel_ref[...]

    s = _dot_t(qn, kn) + _dot_t(qr, kr)
    s = jnp.where(sel != 0, s, NEG)
    m = jnp.max(s, axis=1, keepdims=True)
    p = jnp.exp(s - m)
    l = jnp.sum(p, axis=1, keepdims=True)
    o_out[...] = (_dot(p.astype(jnp.bfloat16), v) / l).astype(jnp.bfloat16)


def _out_body(o_ref, wo_ref, bo_ref, y_out):
    y_out[...] = _dot_t(o_ref[...], wo_ref[...]) + bo_ref[...]


def _row(w):
    return w.reshape(1, -1)


@jax.jit
def kernel(x, mask, params):
    p = params
    x2 = x.reshape(T, IN_DIM)
    x16 = x2.astype(jnp.bfloat16)
    mask2 = mask.reshape(T, T)

    qc = jnp.asarray(_QC_NP)
    qs = jnp.asarray(_QS_NP)
    ic = jnp.asarray(_IC_NP)
    isn = jnp.asarray(_IS_NP)
    seg = jnp.asarray(_SEG_NP)

    wqa16 = p["Wq_a_w"].astype(jnp.bfloat16)
    wqb16 = p["Wq_b_w"].astype(jnp.bfloat16)
    wkvb16 = p["Wkv_b_w"].astype(jnp.bfloat16)
    wo16 = p["Wo_w"].astype(jnp.bfloat16)
    gknv = _row(jnp.tile(jnp.concatenate([p["k_ln_g"], p["v_ln_g"]]), NH))
    bknv = _row(jnp.tile(jnp.concatenate([p["k_ln_b"], p["v_ln_b"]]), NH))

    nbt = T // BT
    full = lambda shape: pl.BlockSpec(shape, lambda i: (0,) * len(shape))
    rowblk = lambda w: pl.BlockSpec((BT, w), lambda i: (i, 0))

    qn, qr, knv, kr, iqf, ikf, iwf = pl.pallas_call(
        _proj_body,
        grid=(nbt,),
        in_specs=[
            rowblk(IN_DIM), rowblk(IN_DIM),
            rowblk(QK_ROPE), rowblk(QK_ROPE),
            rowblk(IRD), rowblk(IRD), full((KV_B, 2 * NH)),
            full((Q_LORA, IN_DIM)), full((1, Q_LORA)),
            full((NH * HD, Q_LORA)), full((1, NH * HD)),
            full((1, NH * HD)), full((1, NH * HD)),
            full((KV_LORA + QK_ROPE, IN_DIM)), full((1, KV_LORA + QK_ROPE)),
            full((1, KV_LORA + QK_ROPE)), full((1, KV_LORA + QK_ROPE)),
            full((KV_B, KV_LORA)), full((1, KV_B)),
            full((1, KV_B)), full((1, KV_B)),
            full((IH * IHD, IN_DIM)), full((1, IH * IHD)),
            full((1, IH * IHD)), full((1, IH * IHD)),
            full((IHD, KV_LORA)), full((1, IHD)),
            full((1, IHD)), full((1, IHD)),
            full((IH, IN_DIM)), full((1, IH)),
            full((1, IH)), full((1, IH)),
        ],
        out_specs=[
            rowblk(NH * QK_NOPE),
            pl.BlockSpec((NH, BT, QK_ROPE), lambda i: (0, i, 0)),
            rowblk(KV_B), rowblk(QK_ROPE),
            rowblk(IH * IHD), rowblk(IHD), rowblk(IH),
        ],
        out_shape=[
            jax.ShapeDtypeStruct((T, NH * QK_NOPE), jnp.bfloat16),
            jax.ShapeDtypeStruct((NH, T, QK_ROPE), jnp.bfloat16),
            jax.ShapeDtypeStruct((T, KV_B), jnp.bfloat16),
            jax.ShapeDtypeStruct((T, QK_ROPE), jnp.bfloat16),
            jax.ShapeDtypeStruct((T, IH * IHD), jnp.float32),
            jax.ShapeDtypeStruct((T, IHD), jnp.float32),
            jax.ShapeDtypeStruct((T, IH), jnp.float32),
        ],
    )(x2, x16, qc, qs, ic, isn, seg,
      wqa16, _row(p["Wq_a_b"]), wqb16, _row(p["Wq_b_b"]),
      _row(p["q_ln_g"]), _row(p["q_ln_b"]),
      p["Wkv_a_w"], _row(p["Wkv_a_b"]), _row(p["kv_ln_g"]), _row(p["kv_ln_b"]),
      wkvb16, _row(p["Wkv_b_b"]), gknv, bknv,
      p["iWq_w"], _row(p["iWq_b"]), _row(p["iq_ln_g"]), _row(p["iq_ln_b"]),
      p["iWk_w"], _row(p["iWk_b"]), _row(p["ik_ln_g"]), _row(p["ik_ln_b"]),
      p["iWw_w"], _row(p["iWw_b"]), _row(p["iw_ln_g"]), _row(p["iw_ln_b"]))

    nbq = T // BQ
    sel = pl.pallas_call(
        _select_body,
        grid=(nbq,),
        in_specs=[
            pl.BlockSpec((BQ, IH * IHD), lambda i: (i, 0)),
            pl.BlockSpec((T, IHD), lambda i: (0, 0)),
            pl.BlockSpec((BQ, IH), lambda i: (i, 0)),
            pl.BlockSpec((BQ, T), lambda i: (i, 0)),
        ],
        out_specs=pl.BlockSpec((BQ, T), lambda i: (i, 0)),
        out_shape=jax.ShapeDtypeStruct((T, T), jnp.int8),
    )(iqf, ikf, iwf, mask2)

    nba = T // BA
    o = pl.pallas_call(
        _attn_body,
        grid=(NH, nba),
        in_specs=[
            pl.BlockSpec((BA, QK_NOPE), lambda h, i: (i, h)),
            pl.BlockSpec((1, BA, QK_ROPE), lambda h, i: (h, i, 0)),
            pl.BlockSpec((T, QK_NOPE), lambda h, i: (0, 2 * h)),
            pl.BlockSpec((T, QK_ROPE), lambda h, i: (0, 0)),
            pl.BlockSpec((T, V_HEAD), lambda h, i: (0, 2 * h + 1)),
            pl.BlockSpec((BA, T), lambda h, i: (i, 0)),
        ],
        out_specs=pl.BlockSpec((BA, V_HEAD), lambda h, i: (i, h)),
        out_shape=jax.ShapeDtypeStruct((T, NH * V_HEAD), jnp.bfloat16),
    )(qn, qr, knv, kr, knv, sel)

    y = pl.pallas_call(
        _out_body,
        grid=(nba,),
        in_specs=[
            pl.BlockSpec((BA, NH * V_HEAD), lambda i: (i, 0)),
            pl.BlockSpec((IN_DIM, NH * V_HEAD), lambda i: (0, 0)),
            pl.BlockSpec((1, IN_DIM), lambda i: (0, 0)),
        ],
        out_specs=pl.BlockSpec((BA, IN_DIM), lambda i: (i, 0)),
        out_shape=jax.ShapeDtypeStruct((T, IN_DIM), jnp.float32),
    )(o, wo16, _row(p["Wo_b"]))

    return y.reshape(1, T, IN_DIM)
